# Initial kernel scaffold; baseline (speedup 1.0000x reference)
import jax
import jax.numpy as jnp
from jax.experimental import pallas as pl


def kernel(x, edge_index, attention, Wz, bz, lz_W, lz_b, Wr, br, lr_W, lr_b, Wh, bh, lh_W, lh_b, lin_W, lin_b):
    raise NotImplementedError("write your pallas kernel here")


# speedup vs baseline: 175.5101x; 175.5101x over previous
import functools

import jax
import jax.numpy as jnp
from jax import lax
from jax.experimental import pallas as pl
from jax.experimental.pallas import tpu as pltpu
from jax.experimental.pallas import tpu_sc as plsc

_B, _N, _F, _T, _OUT = 4, 10000, 128, 12, 32
_E = 160000
_NPAD = 10240
_NCH = 48
_EB = 125
_EROWS = _E // _EB
_RPT = _EROWS // 16
_RPS = _NPAD // 16


def _sc_mesh():
    return plsc.VectorSubcoreMesh(core_axis_name="c", subcore_axis_name="s")


def _deg_body(dst_hbm, ones_hbm, deg_hbm, dst_v, ones_v, shared):
    s = lax.axis_index("s")
    pltpu.sync_copy(ones_hbm, ones_v)
    for k in range(_RPS // 128):
        pltpu.sync_copy(ones_v, shared.at[pl.ds(s * _RPS + k * 128, 128)])
    plsc.subcore_barrier()
    pltpu.sync_copy(dst_hbm.at[pl.ds(s * _RPT, _RPT)], dst_v)

    @pl.loop(0, _RPT)
    def _edge_batch(j):
        pltpu.sync_copy(ones_v.at[pl.ds(0, _EB)], shared.at[dst_v.at[j]],
                        add=True)

    plsc.subcore_barrier()
    pltpu.sync_copy(shared.at[pl.ds(s * _RPS, _RPS)],
                    deg_hbm.at[pl.ds(s * _RPS, _RPS)])


def _k_deg(dstm, ones16):
    f = pl.kernel(
        _deg_body,
        mesh=_sc_mesh(),
        out_type=jax.ShapeDtypeStruct((_NPAD, 16), jnp.float32),
        scratch_types=[
            pltpu.VMEM((_RPT, _EB), jnp.int32),
            pltpu.VMEM((128, 16), jnp.float32),
            pltpu.VMEM_SHARED((_NPAD, 16), jnp.float32),
        ],
    )
    return f(dstm, ones16)


def _p1_body(x_ref, w_ref, degb_ref, out_ref):
    h = jnp.dot(x_ref[0], w_ref[...], preferred_element_type=jnp.float32,
                precision=lax.Precision.HIGHEST)
    out_ref[0] = h * lax.rsqrt(degb_ref[...])


def _p1(xt, wzh, degb):
    bn = 400
    grid = (_NCH, _N // bn)
    return pl.pallas_call(
        _p1_body,
        grid=grid,
        in_specs=[
            pl.BlockSpec((1, bn, _F), lambda c, n: (c, n, 0)),
            pl.BlockSpec((_F, 64), lambda c, n: (0, 0)),
            pl.BlockSpec((bn, 64), lambda c, n: (n, 0)),
        ],
        out_specs=pl.BlockSpec((1, bn, 64), lambda c, n: (c, n, 0)),
        out_shape=jax.ShapeDtypeStruct((_NCH, _NPAD, 64), jnp.float32),
    )(xt, wzh, degb)


def _agg_body(hs_hbm, srcoff_hbm, dst_hbm, zeros_hbm, agg_hbm,
              srcb, dstb, rows0, rows1, zero_v, shared, sem0, sem1):
    cax = lax.axis_index("c")
    s = lax.axis_index("s")
    pltpu.sync_copy(dst_hbm.at[pl.ds(s * _RPT, _RPT)], dstb)
    pltpu.sync_copy(zeros_hbm, zero_v)
    for ci in range(_NCH // 2):
        c = ci * 2 + cax
        pltpu.sync_copy(srcoff_hbm.at[pl.ds(c * _EROWS + s * _RPT, _RPT)],
                        srcb)
        for k in range(_RPS // 128):
            pltpu.sync_copy(zero_v, shared.at[pl.ds(s * _RPS + k * 128, 128)])
        plsc.subcore_barrier()

        @pl.loop(0, _RPT // 2)
        def _edge_batch(jj):
            j0 = jj * 2
            g0 = pltpu.async_copy(hs_hbm.at[srcb.at[j0]], rows0, sem0)
            g1 = pltpu.async_copy(hs_hbm.at[srcb.at[j0 + 1]], rows1, sem1)
            g0.wait()
            pltpu.sync_copy(rows0, shared.at[dstb.at[j0]], add=True)
            g1.wait()
            pltpu.sync_copy(rows1, shared.at[dstb.at[j0 + 1]], add=True)

        plsc.subcore_barrier()
        pltpu.sync_copy(shared.at[pl.ds(s * _RPS, _RPS)],
                        agg_hbm.at[pl.ds(c * _NPAD + s * _RPS, _RPS)])
        plsc.subcore_barrier()


def _k_agg(hs_flat, src_off, dstm, zeros128):
    f = pl.kernel(
        _agg_body,
        mesh=_sc_mesh(),
        compiler_params=pltpu.CompilerParams(use_tc_tiling_on_sc=False),
        out_type=jax.ShapeDtypeStruct((_NCH * _NPAD, 64), jnp.float32),
        scratch_types=[
            pltpu.VMEM((_RPT, _EB), jnp.int32),
            pltpu.VMEM((_RPT, _EB), jnp.int32),
            pltpu.VMEM((_EB, 64), jnp.float32),
            pltpu.VMEM((_EB, 64), jnp.float32),
            pltpu.VMEM((128, 64), jnp.float32),
            pltpu.VMEM_SHARED((_NPAD, 64), jnp.float32),
            pltpu.SemaphoreType.DMA,
            pltpu.SemaphoreType.DMA,
        ],
    )
    return f(hs_flat, src_off, dstm, zeros128)


def _sigmoid(v):
    return 1.0 / (1.0 + jnp.exp(-v))


def _p2_body(agg_ref, hs_ref, degb_ref, att_ref, lzw_ref, lzb_ref,
             lhw_ref, lhb_ref, bz_ref, bh_ref, linw_ref, linb_ref, out_ref):
    av = att_ref[...]
    av = jnp.exp(av - jnp.max(av))
    probs = av / jnp.sum(av)
    dinv = lax.rsqrt(degb_ref[...])
    lzw = lzw_ref[...]
    lhw = lhw_ref[...]
    for b in range(_B):
        hb = jnp.zeros((400, _OUT), jnp.float32)
        for t in range(_T):
            bt = b * _T + t
            v = (agg_ref[bt] + hs_ref[bt]) * dinv
            cz = v[:, :_OUT] + bz_ref[...]
            ch = v[:, _OUT:] + bh_ref[...]
            z = _sigmoid(jnp.dot(cz, lzw, preferred_element_type=jnp.float32,
                                 precision=lax.Precision.HIGHEST)
                         + lzb_ref[...])
            ht = jnp.tanh(jnp.dot(ch, lhw, preferred_element_type=jnp.float32,
                                  precision=lax.Precision.HIGHEST)
                          + lhb_ref[...])
            hb = hb + probs[0, t] * (1.0 - z) * ht
        ob = jnp.dot(jnp.maximum(hb, 0.0), linw_ref[...],
                     preferred_element_type=jnp.float32,
                     precision=lax.Precision.HIGHEST) + linb_ref[...]
        out_ref[b] = ob


def _p2(agg3, hs3, degb, att, lzw0, lzb, lhw0, lhb, bz, bh, linw, linb):
    bn = 400
    grid = (_N // bn,)
    full = lambda shape: pl.BlockSpec(shape, lambda n: tuple(0 for _ in shape))
    return pl.pallas_call(
        _p2_body,
        grid=grid,
        in_specs=[
            pl.BlockSpec((_NCH, bn, 64), lambda n: (0, n, 0)),
            pl.BlockSpec((_NCH, bn, 64), lambda n: (0, n, 0)),
            pl.BlockSpec((bn, 64), lambda n: (n, 0)),
            full((1, _T)),
            full((_OUT, _OUT)),
            full((1, _OUT)),
            full((_OUT, _OUT)),
            full((1, _OUT)),
            full((1, _OUT)),
            full((1, _OUT)),
            full((_OUT, _T)),
            full((1, _T)),
        ],
        out_specs=pl.BlockSpec((_B, bn, _T), lambda n: (0, n, 0)),
        out_shape=jax.ShapeDtypeStruct((_B, _N, _T), jnp.float32),
    )(agg3, hs3, degb, att, lzw0, lzb, lhw0, lhb, bz, bh, linw, linb)


def kernel(x, edge_index, attention, Wz, bz, lz_W, lz_b, Wr, br, lr_W, lr_b,
           Wh, bh, lh_W, lh_b, lin_W, lin_b):
    del Wr, br, lr_W, lr_b
    xt = jnp.transpose(x, (0, 3, 1, 2)).reshape(_B * _T, _N, _F)
    src = edge_index[0]
    dst = edge_index[1]
    dstm = dst.reshape(_EROWS, _EB)
    offs = (jnp.arange(_NCH, dtype=jnp.int32) * _NPAD)[:, None]
    src_off = (src[None, :] + offs).reshape(_NCH * _EROWS, _EB)

    ones16 = jnp.ones((128, 16), jnp.float32)
    zeros128 = jnp.zeros((128, 64), jnp.float32)

    deg = _k_deg(dstm, ones16)
    degb = jnp.broadcast_to(deg[:_N, :1], (_N, 64))

    wzh = jnp.concatenate([Wz, Wh], axis=1)
    hs3 = _p1(xt, wzh, degb)
    hs_flat = hs3.reshape(_NCH * _NPAD, 64)

    agg_flat = _k_agg(hs_flat, src_off, dstm, zeros128)
    agg3 = agg_flat.reshape(_NCH, _NPAD, 64)

    return _p2(agg3, hs3, degb,
               attention.reshape(1, _T),
               lz_W[:_OUT], lz_b.reshape(1, _OUT),
               lh_W[:_OUT], lh_b.reshape(1, _OUT),
               bz.reshape(1, _OUT), bh.reshape(1, _OUT),
               lin_W, lin_b.reshape(1, _T))


# speedup vs baseline: 223.7509x; 1.2749x over previous
import functools

import jax
import jax.numpy as jnp
from jax import lax
from jax.experimental import pallas as pl
from jax.experimental.pallas import tpu as pltpu
from jax.experimental.pallas import tpu_sc as plsc

_B, _N, _F, _T, _OUT = 4, 10000, 128, 12, 32
_E = 160000
_NPAD = 10240
_NCH = 48
_EB = 125
_EROWS = _E // _EB
_RPT = _EROWS // 16
_RPS = _NPAD // 16


def _sc_mesh():
    return plsc.VectorSubcoreMesh(core_axis_name="c", subcore_axis_name="s")


def _deg_body(dst_hbm, ones_hbm, deg_hbm, dst_v, ones_v, shared):
    s = lax.axis_index("s")
    pltpu.sync_copy(ones_hbm, ones_v)
    for k in range(_RPS // 128):
        pltpu.sync_copy(ones_v, shared.at[pl.ds(s * _RPS + k * 128, 128)])
    plsc.subcore_barrier()
    pltpu.sync_copy(dst_hbm.at[pl.ds(s * _RPT, _RPT)], dst_v)

    @pl.loop(0, _RPT)
    def _edge_batch(j):
        pltpu.sync_copy(ones_v.at[pl.ds(0, _EB)], shared.at[dst_v.at[j]],
                        add=True)

    plsc.subcore_barrier()
    pltpu.sync_copy(shared.at[pl.ds(s * _RPS, _RPS)],
                    deg_hbm.at[pl.ds(s * _RPS, _RPS)])


def _k_deg(dstm, ones16):
    f = pl.kernel(
        _deg_body,
        mesh=_sc_mesh(),
        out_type=jax.ShapeDtypeStruct((_NPAD, 16), jnp.float32),
        scratch_types=[
            pltpu.VMEM((_RPT, _EB), jnp.int32),
            pltpu.VMEM((128, 16), jnp.float32),
            pltpu.VMEM_SHARED((_NPAD, 16), jnp.float32),
        ],
    )
    return f(dstm, ones16)


def _p1_body(x_ref, w_ref, degb_ref, out_ref):
    h = jnp.dot(x_ref[0], w_ref[...], preferred_element_type=jnp.float32,
                precision=lax.Precision.HIGHEST)
    out_ref[0] = h * lax.rsqrt(degb_ref[...])


def _p1(xt, wzh, degb):
    bn = 400
    grid = (_NCH, _N // bn)
    return pl.pallas_call(
        _p1_body,
        grid=grid,
        in_specs=[
            pl.BlockSpec((1, bn, _F), lambda c, n: (c, n, 0)),
            pl.BlockSpec((_F, 64), lambda c, n: (0, 0)),
            pl.BlockSpec((bn, 64), lambda c, n: (n, 0)),
        ],
        out_specs=pl.BlockSpec((1, bn, 64), lambda c, n: (c, n, 0)),
        out_shape=jax.ShapeDtypeStruct((_NCH, _NPAD, 64), jnp.float32),
    )(xt, wzh, degb)


_NBUF = 5
_LOOK = 3


def _agg_body(hs_hbm, srcoff_hbm, dst_hbm, zeros_hbm, agg_hbm,
              srcb, dstb, zero_v, shared, *bufsems):
    rows = bufsems[:_NBUF]
    sg = bufsems[_NBUF:2 * _NBUF]
    ss = bufsems[2 * _NBUF:3 * _NBUF]
    cax = lax.axis_index("c")
    s = lax.axis_index("s")
    pltpu.sync_copy(dst_hbm.at[pl.ds(s * _RPT, _RPT)], dstb)
    pltpu.sync_copy(zeros_hbm, zero_v)

    def wait_gather(j, k):
        pltpu.make_async_copy(hs_hbm.at[srcb.at[j]], rows[k], sg[k]).wait()

    def wait_scatter(k):
        pltpu.make_async_copy(rows[k], shared.at[dstb.at[0]], ss[k]).wait()

    for ci in range(_NCH // 2):
        c = ci * 2 + cax
        pltpu.sync_copy(srcoff_hbm.at[pl.ds(c * _EROWS + s * _RPT, _RPT)],
                        srcb)
        for k in range(_RPS // 128):
            pltpu.sync_copy(zero_v, shared.at[pl.ds(s * _RPS + k * 128, 128)])
        plsc.subcore_barrier()

        for k in range(_LOOK):
            pltpu.async_copy(hs_hbm.at[srcb.at[k]], rows[k], sg[k])

        @pl.loop(0, _RPT // _NBUF)
        def _edge_batch(jj):
            j0 = jj * _NBUF
            for k in range(_NBUF):
                j = j0 + k
                wait_gather(j, k)
                pltpu.async_copy(rows[k], shared.at[dstb.at[j]], ss[k],
                                 add=True)
                jf = j + _LOOK
                kf = (k + _LOOK) % _NBUF
                if k + _LOOK >= _NBUF:
                    @pl.when(jf < _RPT)
                    def _issue():
                        wait_scatter(kf)
                        pltpu.async_copy(hs_hbm.at[srcb.at[jf]], rows[kf],
                                         sg[kf])
                else:
                    @pl.when(jf < _RPT)
                    def _issue():
                        @pl.when(jj > 0)
                        def _w():
                            wait_scatter(kf)
                        pltpu.async_copy(hs_hbm.at[srcb.at[jf]], rows[kf],
                                         sg[kf])

        for k in range(_NBUF):
            wait_scatter(k)
        plsc.subcore_barrier()
        pltpu.sync_copy(shared.at[pl.ds(s * _RPS, _RPS)],
                        agg_hbm.at[pl.ds(c * _NPAD + s * _RPS, _RPS)])
        plsc.subcore_barrier()


def _k_agg(hs_flat, src_off, dstm, zeros128):
    f = pl.kernel(
        _agg_body,
        mesh=_sc_mesh(),
        compiler_params=pltpu.CompilerParams(use_tc_tiling_on_sc=False),
        out_type=jax.ShapeDtypeStruct((_NCH * _NPAD, 64), jnp.float32),
        scratch_types=(
            [pltpu.VMEM((_RPT, _EB), jnp.int32),
             pltpu.VMEM((_RPT, _EB), jnp.int32),
             pltpu.VMEM((128, 64), jnp.float32),
             pltpu.VMEM_SHARED((_NPAD, 64), jnp.float32)]
            + [pltpu.VMEM((_EB, 64), jnp.float32)] * _NBUF
            + [pltpu.SemaphoreType.DMA] * (2 * _NBUF)
        ),
    )
    return f(hs_flat, src_off, dstm, zeros128)


def _sigmoid(v):
    return 1.0 / (1.0 + jnp.exp(-v))


def _p2_body(agg_ref, hs_ref, degb_ref, att_ref, lzw_ref, lzb_ref,
             lhw_ref, lhb_ref, bz_ref, bh_ref, linw_ref, linb_ref, out_ref):
    av = att_ref[...]
    av = jnp.exp(av - jnp.max(av))
    probs = av / jnp.sum(av)
    dinv = lax.rsqrt(degb_ref[...])
    lzw = lzw_ref[...]
    lhw = lhw_ref[...]
    for b in range(_B):
        hb = jnp.zeros((400, _OUT), jnp.float32)
        for t in range(_T):
            bt = b * _T + t
            v = (agg_ref[bt] + hs_ref[bt]) * dinv
            cz = v[:, :_OUT] + bz_ref[...]
            ch = v[:, _OUT:] + bh_ref[...]
            z = _sigmoid(jnp.dot(cz, lzw, preferred_element_type=jnp.float32,
                                 precision=lax.Precision.HIGHEST)
                         + lzb_ref[...])
            ht = jnp.tanh(jnp.dot(ch, lhw, preferred_element_type=jnp.float32,
                                  precision=lax.Precision.HIGHEST)
                          + lhb_ref[...])
            hb = hb + probs[0, t] * (1.0 - z) * ht
        ob = jnp.dot(jnp.maximum(hb, 0.0), linw_ref[...],
                     preferred_element_type=jnp.float32,
                     precision=lax.Precision.HIGHEST) + linb_ref[...]
        out_ref[b] = ob


def _p2(agg3, hs3, degb, att, lzw0, lzb, lhw0, lhb, bz, bh, linw, linb):
    bn = 400
    grid = (_N // bn,)
    full = lambda shape: pl.BlockSpec(shape, lambda n: tuple(0 for _ in shape))
    return pl.pallas_call(
        _p2_body,
        grid=grid,
        in_specs=[
            pl.BlockSpec((_NCH, bn, 64), lambda n: (0, n, 0)),
            pl.BlockSpec((_NCH, bn, 64), lambda n: (0, n, 0)),
            pl.BlockSpec((bn, 64), lambda n: (n, 0)),
            full((1, _T)),
            full((_OUT, _OUT)),
            full((1, _OUT)),
            full((_OUT, _OUT)),
            full((1, _OUT)),
            full((1, _OUT)),
            full((1, _OUT)),
            full((_OUT, _T)),
            full((1, _T)),
        ],
        out_specs=pl.BlockSpec((_B, bn, _T), lambda n: (0, n, 0)),
        out_shape=jax.ShapeDtypeStruct((_B, _N, _T), jnp.float32),
    )(agg3, hs3, degb, att, lzw0, lzb, lhw0, lhb, bz, bh, linw, linb)


def kernel(x, edge_index, attention, Wz, bz, lz_W, lz_b, Wr, br, lr_W, lr_b,
           Wh, bh, lh_W, lh_b, lin_W, lin_b):
    del Wr, br, lr_W, lr_b
    xt = jnp.transpose(x, (0, 3, 1, 2)).reshape(_B * _T, _N, _F)
    src = edge_index[0]
    dst = edge_index[1]
    dstm = dst.reshape(_EROWS, _EB)
    offs = (jnp.arange(_NCH, dtype=jnp.int32) * _NPAD)[:, None]
    src_off = (src[None, :] + offs).reshape(_NCH * _EROWS, _EB)

    ones16 = jnp.ones((128, 16), jnp.float32)
    zeros128 = jnp.zeros((128, 64), jnp.float32)

    deg = _k_deg(dstm, ones16)
    degb = jnp.broadcast_to(deg[:_N, :1], (_N, 64))

    wzh = jnp.concatenate([Wz, Wh], axis=1)
    hs3 = _p1(xt, wzh, degb)
    hs_flat = hs3.reshape(_NCH * _NPAD, 64)

    agg_flat = _k_agg(hs_flat, src_off, dstm, zeros128)
    agg3 = agg_flat.reshape(_NCH, _NPAD, 64)

    return _p2(agg3, hs3, degb,
               attention.reshape(1, _T),
               lz_W[:_OUT], lz_b.reshape(1, _OUT),
               lh_W[:_OUT], lh_b.reshape(1, _OUT),
               bz.reshape(1, _OUT), bh.reshape(1, _OUT),
               lin_W, lin_b.reshape(1, _T))


# speedup vs baseline: 260.5482x; 1.1645x over previous
import functools

import jax
import jax.numpy as jnp
from jax import lax
from jax.experimental import pallas as pl
from jax.experimental.pallas import tpu as pltpu
from jax.experimental.pallas import tpu_sc as plsc

_B, _N, _F, _T, _OUT = 4, 10000, 128, 12, 32
_E = 160000
_NPAD = 10240
_NCH = 48
_EB = 125
_EROWS = _E // _EB
_RPT = _EROWS // 16
_RPS = _NPAD // 16


def _sc_mesh():
    return plsc.VectorSubcoreMesh(core_axis_name="c", subcore_axis_name="s")


def _deg_body(dst_hbm, ones_hbm, deg_hbm, dst_v, ones_v, shared):
    s = lax.axis_index("s")
    pltpu.sync_copy(ones_hbm, ones_v)
    for k in range(_RPS // 128):
        pltpu.sync_copy(ones_v, shared.at[pl.ds(s * _RPS + k * 128, 128)])
    plsc.subcore_barrier()
    pltpu.sync_copy(dst_hbm.at[pl.ds(s * _RPT, _RPT)], dst_v)

    @pl.loop(0, _RPT)
    def _edge_batch(j):
        pltpu.sync_copy(ones_v.at[pl.ds(0, _EB)], shared.at[dst_v.at[j]],
                        add=True)

    plsc.subcore_barrier()
    pltpu.sync_copy(shared.at[pl.ds(s * _RPS, _RPS)],
                    deg_hbm.at[pl.ds(s * _RPS, _RPS)])


def _k_deg(dstm, ones16):
    f = pl.kernel(
        _deg_body,
        mesh=_sc_mesh(),
        out_type=jax.ShapeDtypeStruct((_NPAD, 16), jnp.float32),
        scratch_types=[
            pltpu.VMEM((_RPT, _EB), jnp.int32),
            pltpu.VMEM((128, 16), jnp.float32),
            pltpu.VMEM_SHARED((_NPAD, 16), jnp.float32),
        ],
    )
    return f(dstm, ones16)


_P1BN = 1000


def _p1_body(x_ref, w_ref, degb_ref, out_ref):
    xb = x_ref[0].astype(jnp.bfloat16)
    h = jnp.dot(xb, w_ref[...], preferred_element_type=jnp.float32)
    dinv = lax.rsqrt(degb_ref[...])[:, :1]
    out_ref[0] = h * jnp.broadcast_to(dinv, h.shape)


def _p1(x2, w2, degb):
    bn = _P1BN
    grid = (_B, _N // bn)
    return pl.pallas_call(
        _p1_body,
        grid=grid,
        in_specs=[
            pl.BlockSpec((1, bn, _F * _T), lambda b, n: (b, n, 0)),
            pl.BlockSpec((_F * _T, _T * 64), lambda b, n: (0, 0)),
            pl.BlockSpec((bn, 64), lambda b, n: (n, 0)),
        ],
        out_specs=pl.BlockSpec((1, bn, _T * 64), lambda b, n: (b, n, 0)),
        out_shape=jax.ShapeDtypeStruct((_B, _NPAD, _T * 64), jnp.float32),
    )(x2, w2, degb)


_NBUF = 5
_LOOK = 3


def _agg_body(hs_hbm, srcoff_hbm, dst_hbm, zeros_hbm, agg_hbm,
              srcb, dstb, zero_v, shared, *bufsems):
    rows = bufsems[:_NBUF]
    sg = bufsems[_NBUF:2 * _NBUF]
    ss = bufsems[2 * _NBUF:3 * _NBUF]
    cax = lax.axis_index("c")
    s = lax.axis_index("s")
    pltpu.sync_copy(dst_hbm.at[pl.ds(s * _RPT, _RPT)], dstb)
    pltpu.sync_copy(zeros_hbm, zero_v)

    def wait_gather(j, k):
        pltpu.make_async_copy(hs_hbm.at[srcb.at[j]], rows[k], sg[k]).wait()

    def wait_scatter(k):
        pltpu.make_async_copy(rows[k], shared.at[dstb.at[0]], ss[k]).wait()

    for ci in range(_NCH // 2):
        c = ci * 2 + cax
        pltpu.sync_copy(srcoff_hbm.at[pl.ds(c * _EROWS + s * _RPT, _RPT)],
                        srcb)
        for k in range(_RPS // 128):
            pltpu.sync_copy(zero_v, shared.at[pl.ds(s * _RPS + k * 128, 128)])
        plsc.subcore_barrier()

        for k in range(_LOOK):
            pltpu.async_copy(hs_hbm.at[srcb.at[k]], rows[k], sg[k])

        @pl.loop(0, _RPT // _NBUF)
        def _edge_batch(jj):
            j0 = jj * _NBUF
            for k in range(_NBUF):
                j = j0 + k
                wait_gather(j, k)
                pltpu.async_copy(rows[k], shared.at[dstb.at[j]], ss[k],
                                 add=True)
                jf = j + _LOOK
                kf = (k + _LOOK) % _NBUF
                if k + _LOOK >= _NBUF:
                    @pl.when(jf < _RPT)
                    def _issue():
                        wait_scatter(kf)
                        pltpu.async_copy(hs_hbm.at[srcb.at[jf]], rows[kf],
                                         sg[kf])
                else:
                    @pl.when(jf < _RPT)
                    def _issue():
                        @pl.when(jj > 0)
                        def _w():
                            wait_scatter(kf)
                        pltpu.async_copy(hs_hbm.at[srcb.at[jf]], rows[kf],
                                         sg[kf])

        for k in range(_NBUF):
            wait_scatter(k)
        plsc.subcore_barrier()
        pltpu.sync_copy(shared.at[pl.ds(s * _RPS, _RPS)],
                        agg_hbm.at[pl.ds(c * _NPAD + s * _RPS, _RPS)])
        plsc.subcore_barrier()


def _k_agg(hs_flat, src_off, dstm, zeros128):
    f = pl.kernel(
        _agg_body,
        mesh=_sc_mesh(),
        compiler_params=pltpu.CompilerParams(use_tc_tiling_on_sc=False),
        out_type=jax.ShapeDtypeStruct((_NCH * _NPAD, 64), jnp.float32),
        scratch_types=(
            [pltpu.VMEM((_RPT, _EB), jnp.int32),
             pltpu.VMEM((_RPT, _EB), jnp.int32),
             pltpu.VMEM((128, 64), jnp.float32),
             pltpu.VMEM_SHARED((_NPAD, 64), jnp.float32)]
            + [pltpu.VMEM((_EB, 64), jnp.float32)] * _NBUF
            + [pltpu.SemaphoreType.DMA] * (2 * _NBUF)
        ),
    )
    return f(hs_flat, src_off, dstm, zeros128)


def _sigmoid(v):
    return 1.0 / (1.0 + jnp.exp(-v))


def _p2_body(agg_ref, hs_ref, degb_ref, att_ref, lzw_ref, lzb_ref,
             lhw_ref, lhb_ref, bz_ref, bh_ref, linw_ref, linb_ref, out_ref):
    av = att_ref[...]
    av = jnp.exp(av - jnp.max(av))
    probs = av / jnp.sum(av)
    dinv = lax.rsqrt(degb_ref[...])
    lzw = lzw_ref[...]
    lhw = lhw_ref[...]
    for b in range(_B):
        hb = jnp.zeros((400, _OUT), jnp.float32)
        for t in range(_T):
            bt = b * _T + t
            v = (agg_ref[bt] + hs_ref[b, :, t * 64:(t + 1) * 64]) * dinv
            cz = v[:, :_OUT] + bz_ref[...]
            ch = v[:, _OUT:] + bh_ref[...]
            z = _sigmoid(jnp.dot(cz, lzw, preferred_element_type=jnp.float32,
                                 precision=lax.Precision.HIGHEST)
                         + lzb_ref[...])
            ht = jnp.tanh(jnp.dot(ch, lhw, preferred_element_type=jnp.float32,
                                  precision=lax.Precision.HIGHEST)
                          + lhb_ref[...])
            hb = hb + probs[0, t] * (1.0 - z) * ht
        ob = jnp.dot(jnp.maximum(hb, 0.0), linw_ref[...],
                     preferred_element_type=jnp.float32,
                     precision=lax.Precision.HIGHEST) + linb_ref[...]
        out_ref[b] = ob


def _p2(agg3, hs3, degb, att, lzw0, lzb, lhw0, lhb, bz, bh, linw, linb):
    bn = 400
    grid = (_N // bn,)
    full = lambda shape: pl.BlockSpec(shape, lambda n: tuple(0 for _ in shape))
    return pl.pallas_call(
        _p2_body,
        grid=grid,
        in_specs=[
            pl.BlockSpec((_NCH, bn, 64), lambda n: (0, n, 0)),
            pl.BlockSpec((_B, bn, _T * 64), lambda n: (0, n, 0)),
            pl.BlockSpec((bn, 64), lambda n: (n, 0)),
            full((1, _T)),
            full((_OUT, _OUT)),
            full((1, _OUT)),
            full((_OUT, _OUT)),
            full((1, _OUT)),
            full((1, _OUT)),
            full((1, _OUT)),
            full((_OUT, _T)),
            full((1, _T)),
        ],
        out_specs=pl.BlockSpec((_B, bn, _T), lambda n: (0, n, 0)),
        out_shape=jax.ShapeDtypeStruct((_B, _N, _T), jnp.float32),
    )(agg3, hs3, degb, att, lzw0, lzb, lhw0, lhb, bz, bh, linw, linb)


def kernel(x, edge_index, attention, Wz, bz, lz_W, lz_b, Wr, br, lr_W, lr_b,
           Wh, bh, lh_W, lh_b, lin_W, lin_b):
    del Wr, br, lr_W, lr_b
    x2 = x.reshape(_B, _N, _F * _T)
    src = edge_index[0]
    dst = edge_index[1]
    dstm = dst.reshape(_EROWS, _EB)
    bt_b = jnp.arange(_NCH, dtype=jnp.int32) // _T
    bt_t = jnp.arange(_NCH, dtype=jnp.int32) % _T
    offs = (bt_b * _NPAD * _T + bt_t)[:, None]
    src_off = (src[None, :] * _T + offs).reshape(_NCH * _EROWS, _EB)

    ones16 = jnp.ones((128, 16), jnp.float32)
    zeros128 = jnp.zeros((128, 64), jnp.float32)

    deg = _k_deg(dstm, ones16)
    degb = jnp.broadcast_to(deg[:_N, :1], (_N, 64))

    wzh = jnp.concatenate([Wz, Wh], axis=1)
    eye_t = jnp.eye(_T, dtype=jnp.float32)
    w2 = (eye_t[None, :, :, None] * wzh[:, None, None, :]).reshape(
        _F * _T, _T * 64).astype(jnp.bfloat16)
    hs4 = _p1(x2, w2, degb)
    hs_flat = hs4.reshape(_B * _NPAD * _T, 64)

    agg_flat = _k_agg(hs_flat, src_off, dstm, zeros128)
    agg3 = agg_flat.reshape(_NCH, _NPAD, 64)

    return _p2(agg3, hs4, degb,
               attention.reshape(1, _T),
               lz_W[:_OUT], lz_b.reshape(1, _OUT),
               lh_W[:_OUT], lh_b.reshape(1, _OUT),
               bz.reshape(1, _OUT), bh.reshape(1, _OUT),
               lin_W, lin_b.reshape(1, _T))
